# baseline (speedup 1.0000x reference)
import functools

import jax
import jax.numpy as jnp
from jax import lax
from jax.experimental import pallas as pl
from jax.experimental.pallas import tpu as pltpu
from jax.experimental.pallas import tpu_sc as plsc

TOPK = 9
GRP = 128


def _tc1_body(tref, pref, gsel_o, *, npad: int, tb: int, gpad: int):
    tx = tref[:, 0:1]
    ty = tref[:, 1:2]
    px = pref[0:1, :]
    py = pref[1:2, :]
    dx = tx - px
    dy = ty - py
    d2 = dx * dx + dy * dy
    ng = npad // GRP
    g = jnp.sqrt(jnp.min(d2.reshape(tb, ng, GRP), axis=2))
    pad = jnp.full((tb, gpad - ng), jnp.inf, jnp.float32)
    g = jnp.concatenate([g, pad], axis=1)
    giota = lax.broadcasted_iota(jnp.int32, (tb, gpad), 1)
    cols = []
    for _ in range(TOPK):
        m = jnp.min(g, axis=1, keepdims=True)
        idxv = jnp.min(jnp.where(g == m, giota, gpad), axis=1, keepdims=True)
        g = jnp.where(giota == idxv, jnp.inf, g)
        cols.append(idxv)
    cols += [jnp.zeros((tb, 1), jnp.int32)] * (16 - TOPK)
    gsel_o[...] = jnp.concatenate(cols, axis=1)


def _tc2_body(tref, pxref, pyref, gselref, idx_o, *, tb: int, npad: int):
    tx = tref[:, 0:1]
    ty = tref[:, 1:2]
    pxc = pxref[...]
    pyc = pyref[...]
    dx = tx - pxc
    dy = ty - pyc
    d = jnp.sqrt(dx * dx + dy * dy)
    gsel = gselref[...]
    width = 16 * GRP
    lanes = lax.broadcasted_iota(jnp.int32, (tb, width), 1)
    seg = lanes // GRP
    gidx = jnp.take_along_axis(gsel, seg, axis=1) * GRP + lanes % GRP
    d = jnp.where(lanes < TOPK * GRP, d, jnp.inf)
    cols = []
    for _ in range(TOPK):
        m = jnp.min(d, axis=1, keepdims=True)
        wi = jnp.min(jnp.where(d == m, gidx, npad), axis=1, keepdims=True)
        d = jnp.where(gidx == wi, jnp.inf, d)
        cols.append(wi)
    idx_o[...] = jnp.concatenate(cols, axis=1)


def _sc_compact(nt: int, ng: int):
    info = plsc.get_sparse_core_info()
    nc, ns = info.num_cores, info.num_subcores
    nw = nc * ns
    rows_w = nt // nw
    mesh = plsc.VectorSubcoreMesh(core_axis_name="c", subcore_axis_name="s")

    @functools.partial(
        pl.kernel, mesh=mesh,
        out_type=[jax.ShapeDtypeStruct((nt * 16, GRP), jnp.float32),
                  jax.ShapeDtypeStruct((nt * 16, GRP), jnp.float32)],
        scratch_types=[
            pltpu.VMEM((rows_w, 16), jnp.int32),
            pltpu.VMEM((rows_w * 16,), jnp.int32),
            pltpu.VMEM((rows_w * 16, GRP), jnp.float32),
            pltpu.VMEM((rows_w * 16, GRP), jnp.float32),
            pltpu.SemaphoreType.DMA,
        ],
    )
    def sc_fn(gsel_hbm, pxg_hbm, pyg_hbm, pxc_hbm, pyc_hbm,
              gsel_v, sidx_v, segx_v, segy_v, sem):
        c = lax.axis_index("c")
        s = lax.axis_index("s")
        wid = s * nc + c
        base = wid * rows_w
        pltpu.sync_copy(gsel_hbm.at[pl.ds(base, rows_w)], gsel_v)

        def row(r, _):
            gv = gsel_v[r]
            sidx_v[pl.ds(r * 16, 16)] = gv
            return 0

        lax.fori_loop(0, rows_w, row, 0, unroll=False)
        pltpu.async_copy(pxg_hbm.at[sidx_v], segx_v, sem).wait()
        pltpu.async_copy(pyg_hbm.at[sidx_v], segy_v, sem).wait()
        pltpu.sync_copy(segx_v, pxc_hbm.at[pl.ds(base * 16, rows_w * 16)])
        pltpu.sync_copy(segy_v, pyc_hbm.at[pl.ds(base * 16, rows_w * 16)])

    return sc_fn


def _sc_epilogue(nt: int, npad: int):
    info = plsc.get_sparse_core_info()
    nc, ns = info.num_cores, info.num_subcores
    nw = nc * ns
    rows_w = nt // nw
    gidx = rows_w * TOPK
    mesh = plsc.VectorSubcoreMesh(core_axis_name="c", subcore_axis_name="s")

    @functools.partial(
        pl.kernel, mesh=mesh,
        out_type=jax.ShapeDtypeStruct((nt, 128), jnp.float32),
        scratch_types=[
            pltpu.VMEM((gidx,), jnp.int32),
            pltpu.VMEM((gidx, 128), jnp.float32),
            pltpu.VMEM((rows_w, 128), jnp.float32),
            pltpu.VMEM((rows_w, 128), jnp.float32),
            pltpu.SemaphoreType.DMA,
        ],
    )
    def sc_fn(idx_hbm, boxes_hbm, tgt_hbm, out_hbm,
              idx_v, rows_v, tv, ov, sem):
        c = lax.axis_index("c")
        s = lax.axis_index("s")
        wid = s * nc + c
        base = wid * rows_w
        pltpu.sync_copy(idx_hbm.at[pl.ds(wid * gidx, gidx)], idx_v)
        pltpu.async_copy(boxes_hbm.at[idx_v], rows_v, sem).wait()
        pltpu.sync_copy(tgt_hbm.at[pl.ds(base, rows_w)], tv)

        lane = lax.iota(jnp.int32, 16)
        valid = lane < TOPK
        comp = [((lane * 0) + cc).reshape(16, 1) for cc in range(4)]
        dnums = lax.GatherDimensionNumbers(
            offset_dims=(), collapsed_slice_dims=(0,), start_index_map=(0,))

        def permute(vec, idx):
            return lax.gather(vec, idx, dnums, (1,),
                              mode=lax.GatherScatterMode.PROMISE_IN_BOUNDS)

        def splat(vec, cc):
            return permute(vec, comp[cc])

        perms = [((lane + sh) % 16).reshape(16, 1) for sh in (8, 4, 2, 1)]

        def lanesum(vec):
            for p in perms:
                vec = vec + permute(vec, p)
            return vec

        zf = jnp.zeros((16,), jnp.float32)
        for r in range(rows_w):
            cx, cy, cw, ch = zf, zf, zf, zf
            for k in range(TOPK):
                rv = rows_v[r * TOPK + k, 0:16]
                sel = lane == k
                cx = jnp.where(sel, splat(rv, 0), cx)
                cy = jnp.where(sel, splat(rv, 1), cy)
                cw = jnp.where(sel, splat(rv, 2), cw)
                ch = jnp.where(sel, splat(rv, 3), ch)
            trow = tv[r, 0:16]
            tx = splat(trow, 0)
            ty = splat(trow, 1)
            tw = splat(trow, 2)
            th = splat(trow, 3)

            b1_x1 = tx - tw / 2
            b1_x2 = tx + tw / 2
            b1_y1 = ty - th / 2
            b1_y2 = ty + th / 2
            b2_x1 = cx - cw / 2
            b2_x2 = cx + cw / 2
            b2_y1 = cy - ch / 2
            b2_y2 = cy + ch / 2
            iw = jnp.maximum(jnp.minimum(b1_x2, b2_x2) - jnp.maximum(b1_x1, b2_x1), 0.0)
            ih = jnp.maximum(jnp.minimum(b1_y2, b2_y2) - jnp.maximum(b1_y1, b2_y1), 0.0)
            inter = iw * ih
            w1 = b1_x2 - b1_x1
            h1 = b1_y2 - b1_y1
            w2 = b2_x2 - b2_x1
            h2 = b2_y2 - b2_y1
            union = w1 * h1 + 1e-16 + w2 * h2 - inter
            iou = inter / union
            cw_e = jnp.maximum(b1_x2, b2_x2) - jnp.minimum(b1_x1, b2_x1)
            ch_e = jnp.maximum(b1_y2, b2_y2) - jnp.minimum(b1_y1, b2_y1)
            c_area = cw_e * ch_e + 1e-16
            giou = iou - (c_area - union) / c_area

            g0 = jnp.where(valid, giou, 0.0)
            mu = lanesum(g0) / TOPK
            cdev = jnp.where(valid, giou - mu, 0.0)
            var = lanesum(cdev * cdev) / (TOPK - 1)
            thr = mu + var
            maskv = giou > thr

            ov[r, 0:16] = jnp.where(maskv, cx, 0.0)
            ov[r, 16:32] = jnp.where(maskv, cy, 0.0)
            ov[r, 32:48] = jnp.where(maskv, cw, 0.0)
            ov[r, 48:64] = jnp.where(maskv, ch, 0.0)
            ov[r, 64:80] = giou
            ov[r, 80:96] = jnp.where(maskv, 1.0, 0.0)
            ov[r, 96:112] = zf
            ov[r, 112:128] = zf

        pltpu.sync_copy(ov, out_hbm.at[pl.ds(base, rows_w)])

    return sc_fn


@functools.partial(jax.jit, static_argnames=("interpret",))
def kernel(p_boxes, target, interpret=False):
    n = p_boxes.shape[0]
    nt = target.shape[0]
    npad = ((n + GRP - 1) // GRP) * GRP
    ng = npad // GRP
    gpad = ((ng + 127) // 128) * 128
    tb = 64

    pc = p_boxes[:, 2:6]
    px = jnp.pad(pc[:, 0], (0, npad - n), constant_values=1e9)
    py = jnp.pad(pc[:, 1], (0, npad - n), constant_values=1e9)
    pref = jnp.stack([px, py], axis=0)
    tref = target[:, 2:6]

    gsel = pl.pallas_call(
        functools.partial(_tc1_body, npad=npad, tb=tb, gpad=gpad),
        grid=(nt // tb,),
        in_specs=[
            pl.BlockSpec((tb, 4), lambda i: (i, 0)),
            pl.BlockSpec((2, npad), lambda i: (0, 0)),
        ],
        out_specs=pl.BlockSpec((tb, 16), lambda i: (i, 0)),
        out_shape=jax.ShapeDtypeStruct((nt, 16), jnp.int32),
        compiler_params=pltpu.CompilerParams(
            dimension_semantics=("arbitrary",)),
        interpret=interpret,
    )(tref, pref)

    pxg = px.reshape(ng, GRP)
    pyg = py.reshape(ng, GRP)
    pxc, pyc = _sc_compact(nt, ng)(gsel, pxg, pyg)

    idx9 = pl.pallas_call(
        functools.partial(_tc2_body, tb=tb, npad=npad),
        grid=(nt // tb,),
        in_specs=[
            pl.BlockSpec((tb, 4), lambda i: (i, 0)),
            pl.BlockSpec((tb, 16 * GRP), lambda i: (i, 0)),
            pl.BlockSpec((tb, 16 * GRP), lambda i: (i, 0)),
            pl.BlockSpec((tb, 16), lambda i: (i, 0)),
        ],
        out_specs=pl.BlockSpec((tb, TOPK), lambda i: (i, 0)),
        out_shape=jax.ShapeDtypeStruct((nt, TOPK), jnp.int32),
        compiler_params=pltpu.CompilerParams(
            dimension_semantics=("arbitrary",)),
        interpret=interpret,
    )(tref, pxc.reshape(nt, 16 * GRP), pyc.reshape(nt, 16 * GRP), gsel)

    boxes128 = jnp.pad(pc, ((0, npad - n), (0, 124)))
    tgt128 = jnp.pad(tref, ((0, 0), (0, 124)))
    idx_flat = idx9.reshape(-1)

    out = _sc_epilogue(nt, npad)(idx_flat, boxes128, tgt128)

    pos = jnp.stack([out[:, 0:TOPK], out[:, 16:16 + TOPK],
                     out[:, 32:32 + TOPK], out[:, 48:48 + TOPK]], axis=-1)
    return pos, out[:, 64:64 + TOPK], out[:, 80:80 + TOPK].astype(bool)


# speedup vs baseline: 2.6563x; 2.6563x over previous
import functools

import jax
import jax.numpy as jnp
from jax import lax
from jax.experimental import pallas as pl
from jax.experimental.pallas import tpu as pltpu
from jax.experimental.pallas import tpu_sc as plsc

TOPK = 9
GRP = 128


def _tc1_body(tref, pref, gsel_o, *, npad: int, tb: int, gpad: int):
    tx = tref[:, 0:1]
    ty = tref[:, 1:2]
    px = pref[0:1, :]
    py = pref[1:2, :]
    dx = tx - px
    dy = ty - py
    d2 = dx * dx + dy * dy
    ng = npad // GRP
    g = jnp.sqrt(jnp.min(d2.reshape(tb, ng, GRP), axis=2))
    pad = jnp.full((tb, gpad - ng), jnp.inf, jnp.float32)
    g = jnp.concatenate([g, pad], axis=1)
    giota = lax.broadcasted_iota(jnp.int32, (tb, gpad), 1)
    cols = []
    for _ in range(TOPK):
        m = jnp.min(g, axis=1, keepdims=True)
        idxv = jnp.min(jnp.where(g == m, giota, gpad), axis=1, keepdims=True)
        g = jnp.where(giota == idxv, jnp.inf, g)
        cols.append(idxv)
    cols += [jnp.zeros((tb, 1), jnp.int32)] * (16 - TOPK)
    gsel_o[...] = jnp.concatenate(cols, axis=1)


def _tc2_body(tref, pxref, pyref, gselref, idx_o, *, tb: int, npad: int):
    tx = tref[:, 0:1]
    ty = tref[:, 1:2]
    pxc = pxref[...]
    pyc = pyref[...]
    dx = tx - pxc
    dy = ty - pyc
    d = jnp.sqrt(dx * dx + dy * dy)
    gsel = gselref[...]
    width = 16 * GRP
    lanes = lax.broadcasted_iota(jnp.int32, (tb, width), 1)
    seg = lanes // GRP
    gidx = jnp.take_along_axis(gsel, seg, axis=1) * GRP + lanes % GRP
    d = jnp.where(lanes < TOPK * GRP, d, jnp.inf)
    cols = []
    for _ in range(TOPK):
        m = jnp.min(d, axis=1, keepdims=True)
        wi = jnp.min(jnp.where(d == m, gidx, npad), axis=1, keepdims=True)
        d = jnp.where(gidx == wi, jnp.inf, d)
        cols.append(wi)
    idx_o[...] = jnp.concatenate(cols, axis=1)


def _sc_compact(nt: int, ng: int):
    info = plsc.get_sparse_core_info()
    nc, ns = info.num_cores, info.num_subcores
    nw = nc * ns
    rows_w = nt // nw
    mesh = plsc.VectorSubcoreMesh(core_axis_name="c", subcore_axis_name="s")

    @functools.partial(
        pl.kernel, mesh=mesh,
        out_type=[jax.ShapeDtypeStruct((nt * 16, GRP), jnp.float32),
                  jax.ShapeDtypeStruct((nt * 16, GRP), jnp.float32)],
        scratch_types=[
            pltpu.VMEM((rows_w, 16), jnp.int32),
            pltpu.VMEM((rows_w * 16,), jnp.int32),
            pltpu.VMEM((rows_w * 16, GRP), jnp.float32),
            pltpu.VMEM((rows_w * 16, GRP), jnp.float32),
            pltpu.SemaphoreType.DMA,
        ],
    )
    def sc_fn(gsel_hbm, pxg_hbm, pyg_hbm, pxc_hbm, pyc_hbm,
              gsel_v, sidx_v, segx_v, segy_v, sem):
        c = lax.axis_index("c")
        s = lax.axis_index("s")
        wid = s * nc + c
        base = wid * rows_w
        pltpu.sync_copy(gsel_hbm.at[pl.ds(base, rows_w)], gsel_v)

        def row(r, _):
            gv = gsel_v[r]
            sidx_v[pl.ds(r * 16, 16)] = wid * ng + gv
            return 0

        lax.fori_loop(0, rows_w, row, 0, unroll=False)
        pltpu.async_copy(pxg_hbm.at[sidx_v], segx_v, sem).wait()
        pltpu.async_copy(pyg_hbm.at[sidx_v], segy_v, sem).wait()
        pltpu.sync_copy(segx_v, pxc_hbm.at[pl.ds(base * 16, rows_w * 16)])
        pltpu.sync_copy(segy_v, pyc_hbm.at[pl.ds(base * 16, rows_w * 16)])

    return sc_fn


def _sc_epilogue(nt: int, npad: int):
    info = plsc.get_sparse_core_info()
    nc, ns = info.num_cores, info.num_subcores
    nw = nc * ns
    rows_w = nt // nw
    gidx = rows_w * TOPK
    mesh = plsc.VectorSubcoreMesh(core_axis_name="c", subcore_axis_name="s")

    @functools.partial(
        pl.kernel, mesh=mesh,
        out_type=jax.ShapeDtypeStruct((nt, 128), jnp.float32),
        scratch_types=[
            pltpu.VMEM((gidx,), jnp.int32),
            pltpu.VMEM((gidx, 128), jnp.float32),
            pltpu.VMEM((rows_w, 128), jnp.float32),
            pltpu.VMEM((rows_w, 128), jnp.float32),
            pltpu.SemaphoreType.DMA,
        ],
    )
    def sc_fn(idx_hbm, boxes_hbm, tgt_hbm, out_hbm,
              idx_v, rows_v, tv, ov, sem):
        c = lax.axis_index("c")
        s = lax.axis_index("s")
        wid = s * nc + c
        base = wid * rows_w
        pltpu.sync_copy(idx_hbm.at[pl.ds(wid * gidx, gidx)], idx_v)
        pltpu.async_copy(boxes_hbm.at[idx_v], rows_v, sem).wait()
        pltpu.sync_copy(tgt_hbm.at[pl.ds(base, rows_w)], tv)

        lane = lax.iota(jnp.int32, 16)
        valid = lane < TOPK
        comp = [((lane * 0) + cc).reshape(16, 1) for cc in range(4)]
        dnums = lax.GatherDimensionNumbers(
            offset_dims=(), collapsed_slice_dims=(0,), start_index_map=(0,))

        def permute(vec, idx):
            return lax.gather(vec, idx, dnums, (1,),
                              mode=lax.GatherScatterMode.PROMISE_IN_BOUNDS)

        def splat(vec, cc):
            return permute(vec, comp[cc])

        perms = [((lane + sh) % 16).reshape(16, 1) for sh in (8, 4, 2, 1)]

        def lanesum(vec):
            for p in perms:
                vec = vec + permute(vec, p)
            return vec

        zf = jnp.zeros((16,), jnp.float32)
        for r in range(rows_w):
            cx, cy, cw, ch = zf, zf, zf, zf
            for k in range(TOPK):
                rv = rows_v[r * TOPK + k, 0:16]
                sel = lane == k
                cx = jnp.where(sel, splat(rv, 0), cx)
                cy = jnp.where(sel, splat(rv, 1), cy)
                cw = jnp.where(sel, splat(rv, 2), cw)
                ch = jnp.where(sel, splat(rv, 3), ch)
            trow = tv[r, 0:16]
            tx = splat(trow, 0)
            ty = splat(trow, 1)
            tw = splat(trow, 2)
            th = splat(trow, 3)

            b1_x1 = tx - tw / 2
            b1_x2 = tx + tw / 2
            b1_y1 = ty - th / 2
            b1_y2 = ty + th / 2
            b2_x1 = cx - cw / 2
            b2_x2 = cx + cw / 2
            b2_y1 = cy - ch / 2
            b2_y2 = cy + ch / 2
            iw = jnp.maximum(jnp.minimum(b1_x2, b2_x2) - jnp.maximum(b1_x1, b2_x1), 0.0)
            ih = jnp.maximum(jnp.minimum(b1_y2, b2_y2) - jnp.maximum(b1_y1, b2_y1), 0.0)
            inter = iw * ih
            w1 = b1_x2 - b1_x1
            h1 = b1_y2 - b1_y1
            w2 = b2_x2 - b2_x1
            h2 = b2_y2 - b2_y1
            union = w1 * h1 + 1e-16 + w2 * h2 - inter
            iou = inter / union
            cw_e = jnp.maximum(b1_x2, b2_x2) - jnp.minimum(b1_x1, b2_x1)
            ch_e = jnp.maximum(b1_y2, b2_y2) - jnp.minimum(b1_y1, b2_y1)
            c_area = cw_e * ch_e + 1e-16
            giou = iou - (c_area - union) / c_area

            g0 = jnp.where(valid, giou, 0.0)
            mu = lanesum(g0) / TOPK
            cdev = jnp.where(valid, giou - mu, 0.0)
            var = lanesum(cdev * cdev) / (TOPK - 1)
            thr = mu + var
            maskv = giou > thr

            ov[r, 0:16] = jnp.where(maskv, cx, 0.0)
            ov[r, 16:32] = jnp.where(maskv, cy, 0.0)
            ov[r, 32:48] = jnp.where(maskv, cw, 0.0)
            ov[r, 48:64] = jnp.where(maskv, ch, 0.0)
            ov[r, 64:80] = giou
            ov[r, 80:96] = jnp.where(maskv, 1.0, 0.0)
            ov[r, 96:112] = zf
            ov[r, 112:128] = zf

        pltpu.sync_copy(ov, out_hbm.at[pl.ds(base, rows_w)])

    return sc_fn


@functools.partial(jax.jit, static_argnames=("interpret",))
def kernel(p_boxes, target, interpret=False):
    n = p_boxes.shape[0]
    nt = target.shape[0]
    npad = ((n + GRP - 1) // GRP) * GRP
    ng = npad // GRP
    gpad = ((ng + 127) // 128) * 128
    tb = 64

    pc = p_boxes[:, 2:6]
    px = jnp.pad(pc[:, 0], (0, npad - n), constant_values=1e9)
    py = jnp.pad(pc[:, 1], (0, npad - n), constant_values=1e9)
    pref = jnp.stack([px, py], axis=0)
    tref = target[:, 2:6]

    gsel = pl.pallas_call(
        functools.partial(_tc1_body, npad=npad, tb=tb, gpad=gpad),
        grid=(nt // tb,),
        in_specs=[
            pl.BlockSpec((tb, 4), lambda i: (i, 0)),
            pl.BlockSpec((2, npad), lambda i: (0, 0)),
        ],
        out_specs=pl.BlockSpec((tb, 16), lambda i: (i, 0)),
        out_shape=jax.ShapeDtypeStruct((nt, 16), jnp.int32),
        compiler_params=pltpu.CompilerParams(
            dimension_semantics=("arbitrary",)),
        interpret=interpret,
    )(tref, pref)

    pxg = jnp.tile(px.reshape(ng, GRP), (32, 1))
    pyg = jnp.tile(py.reshape(ng, GRP), (32, 1))
    pxc, pyc = _sc_compact(nt, ng)(gsel, pxg, pyg)

    idx9 = pl.pallas_call(
        functools.partial(_tc2_body, tb=tb, npad=npad),
        grid=(nt // tb,),
        in_specs=[
            pl.BlockSpec((tb, 4), lambda i: (i, 0)),
            pl.BlockSpec((tb, 16 * GRP), lambda i: (i, 0)),
            pl.BlockSpec((tb, 16 * GRP), lambda i: (i, 0)),
            pl.BlockSpec((tb, 16), lambda i: (i, 0)),
        ],
        out_specs=pl.BlockSpec((tb, TOPK), lambda i: (i, 0)),
        out_shape=jax.ShapeDtypeStruct((nt, TOPK), jnp.int32),
        compiler_params=pltpu.CompilerParams(
            dimension_semantics=("arbitrary",)),
        interpret=interpret,
    )(tref, pxc.reshape(nt, 16 * GRP), pyc.reshape(nt, 16 * GRP), gsel)

    boxes128 = jnp.pad(pc, ((0, npad - n), (0, 124)))
    tgt128 = jnp.pad(tref, ((0, 0), (0, 124)))
    idx_flat = idx9.reshape(-1)

    out = _sc_epilogue(nt, npad)(idx_flat, boxes128, tgt128)

    pos = jnp.stack([out[:, 0:TOPK], out[:, 16:16 + TOPK],
                     out[:, 32:32 + TOPK], out[:, 48:48 + TOPK]], axis=-1)
    return pos, out[:, 64:64 + TOPK], out[:, 80:80 + TOPK].astype(bool)


# speedup vs baseline: 2.7150x; 1.0221x over previous
import functools

import jax
import jax.numpy as jnp
from jax import lax
from jax.experimental import pallas as pl
from jax.experimental.pallas import tpu as pltpu
from jax.experimental.pallas import tpu_sc as plsc

TOPK = 9
GRP = 128


def _tc1_body(tref, pref, gsel_o, *, npad: int, tb: int, gpad: int):
    tx = tref[:, 0:1]
    ty = tref[:, 1:2]
    px = pref[0:1, :]
    py = pref[1:2, :]
    dx = tx - px
    dy = ty - py
    d2 = dx * dx + dy * dy
    ng = npad // GRP
    g = jnp.sqrt(jnp.min(d2.reshape(tb, ng, GRP), axis=2))
    pad = jnp.full((tb, gpad - ng), jnp.inf, jnp.float32)
    g = jnp.concatenate([g, pad], axis=1)
    giota = lax.broadcasted_iota(jnp.int32, (tb, gpad), 1)
    cols = []
    for _ in range(TOPK):
        m = jnp.min(g, axis=1, keepdims=True)
        idxv = jnp.min(jnp.where(g == m, giota, gpad), axis=1, keepdims=True)
        g = jnp.where(giota == idxv, jnp.inf, g)
        cols.append(idxv)
    cols += [jnp.zeros((tb, 1), jnp.int32)] * (16 - TOPK)
    gsel_o[...] = jnp.concatenate(cols, axis=1)


def _tc2_body(tref, pxref, pyref, gselref, idx_o, *, tb: int, npad: int):
    tx = tref[:, 0:1]
    ty = tref[:, 1:2]
    pxc = pxref[...]
    pyc = pyref[...]
    dx = tx - pxc
    dy = ty - pyc
    d = jnp.sqrt(dx * dx + dy * dy)
    gsel = gselref[...]
    width = TOPK * GRP
    lanes = lax.broadcasted_iota(jnp.int32, (tb, width), 1)
    seg = lanes // GRP
    gidx = jnp.take_along_axis(gsel, seg, axis=1) * GRP + lanes % GRP
    cols = []
    for _ in range(TOPK):
        m = jnp.min(d, axis=1, keepdims=True)
        wi = jnp.min(jnp.where(d == m, gidx, npad), axis=1, keepdims=True)
        d = jnp.where(gidx == wi, jnp.inf, d)
        cols.append(wi)
    idx_o[...] = jnp.concatenate(cols, axis=1)


def _sc_compact(nt: int, ng: int):
    info = plsc.get_sparse_core_info()
    nc, ns = info.num_cores, info.num_subcores
    nw = nc * ns
    rows_w = nt // nw
    mesh = plsc.VectorSubcoreMesh(core_axis_name="c", subcore_axis_name="s")

    @functools.partial(
        pl.kernel, mesh=mesh,
        out_type=[jax.ShapeDtypeStruct((nt * TOPK, GRP), jnp.float32),
                  jax.ShapeDtypeStruct((nt * TOPK, GRP), jnp.float32)],
        scratch_types=[
            pltpu.VMEM((rows_w, 16), jnp.int32),
            pltpu.VMEM((rows_w * TOPK + 16,), jnp.int32),
            pltpu.VMEM((rows_w * TOPK + 16, GRP), jnp.float32),
            pltpu.VMEM((rows_w * TOPK + 16, GRP), jnp.float32),
            pltpu.SemaphoreType.DMA,
        ],
    )
    def sc_fn(gsel_hbm, pxg_hbm, pyg_hbm, pxc_hbm, pyc_hbm,
              gsel_v, sidx_v, segx_v, segy_v, sem):
        c = lax.axis_index("c")
        s = lax.axis_index("s")
        wid = s * nc + c
        base = wid * rows_w
        pltpu.sync_copy(gsel_hbm.at[pl.ds(base, rows_w)], gsel_v)

        def row(r, _):
            gv = gsel_v[r] + wid * ng
            sidx_v[pl.ds(r * TOPK, 16)] = gv
            return 0

        lax.fori_loop(0, rows_w, row, 0, unroll=False)
        pltpu.async_copy(pxg_hbm.at[sidx_v], segx_v, sem).wait()
        pltpu.async_copy(pyg_hbm.at[sidx_v], segy_v, sem).wait()
        pltpu.sync_copy(segx_v.at[pl.ds(0, rows_w * TOPK)],
                        pxc_hbm.at[pl.ds(base * TOPK, rows_w * TOPK)])
        pltpu.sync_copy(segy_v.at[pl.ds(0, rows_w * TOPK)],
                        pyc_hbm.at[pl.ds(base * TOPK, rows_w * TOPK)])

    return sc_fn


def _sc_epilogue(nt: int, npad: int):
    info = plsc.get_sparse_core_info()
    nc, ns = info.num_cores, info.num_subcores
    nw = nc * ns
    rows_w = nt // nw
    gidx = rows_w * TOPK
    mesh = plsc.VectorSubcoreMesh(core_axis_name="c", subcore_axis_name="s")

    @functools.partial(
        pl.kernel, mesh=mesh,
        out_type=jax.ShapeDtypeStruct((nt, 128), jnp.float32),
        scratch_types=[
            pltpu.VMEM((gidx,), jnp.int32),
            pltpu.VMEM((gidx, 128), jnp.float32),
            pltpu.VMEM((rows_w, 128), jnp.float32),
            pltpu.VMEM((rows_w, 128), jnp.float32),
            pltpu.SemaphoreType.DMA,
        ],
    )
    def sc_fn(idx_hbm, boxes_hbm, tgt_hbm, out_hbm,
              idx_v, rows_v, tv, ov, sem):
        c = lax.axis_index("c")
        s = lax.axis_index("s")
        wid = s * nc + c
        base = wid * rows_w
        pltpu.sync_copy(idx_hbm.at[pl.ds(wid * gidx, gidx)], idx_v)
        pltpu.async_copy(boxes_hbm.at[idx_v], rows_v, sem).wait()
        pltpu.sync_copy(tgt_hbm.at[pl.ds(base, rows_w)], tv)

        lane = lax.iota(jnp.int32, 16)
        valid = lane < TOPK
        comp = [((lane * 0) + cc).reshape(16, 1) for cc in range(4)]
        dnums = lax.GatherDimensionNumbers(
            offset_dims=(), collapsed_slice_dims=(0,), start_index_map=(0,))

        def permute(vec, idx):
            return lax.gather(vec, idx, dnums, (1,),
                              mode=lax.GatherScatterMode.PROMISE_IN_BOUNDS)

        def splat(vec, cc):
            return permute(vec, comp[cc])

        perms = [((lane + sh) % 16).reshape(16, 1) for sh in (8, 4, 2, 1)]

        def lanesum(vec):
            for p in perms:
                vec = vec + permute(vec, p)
            return vec

        zf = jnp.zeros((16,), jnp.float32)
        for r in range(rows_w):
            cx, cy, cw, ch = zf, zf, zf, zf
            for k in range(TOPK):
                rv = rows_v[r * TOPK + k, 0:16]
                sel = lane == k
                cx = jnp.where(sel, splat(rv, 0), cx)
                cy = jnp.where(sel, splat(rv, 1), cy)
                cw = jnp.where(sel, splat(rv, 2), cw)
                ch = jnp.where(sel, splat(rv, 3), ch)
            trow = tv[r, 0:16]
            tx = splat(trow, 0)
            ty = splat(trow, 1)
            tw = splat(trow, 2)
            th = splat(trow, 3)

            b1_x1 = tx - tw / 2
            b1_x2 = tx + tw / 2
            b1_y1 = ty - th / 2
            b1_y2 = ty + th / 2
            b2_x1 = cx - cw / 2
            b2_x2 = cx + cw / 2
            b2_y1 = cy - ch / 2
            b2_y2 = cy + ch / 2
            iw = jnp.maximum(jnp.minimum(b1_x2, b2_x2) - jnp.maximum(b1_x1, b2_x1), 0.0)
            ih = jnp.maximum(jnp.minimum(b1_y2, b2_y2) - jnp.maximum(b1_y1, b2_y1), 0.0)
            inter = iw * ih
            w1 = b1_x2 - b1_x1
            h1 = b1_y2 - b1_y1
            w2 = b2_x2 - b2_x1
            h2 = b2_y2 - b2_y1
            union = w1 * h1 + 1e-16 + w2 * h2 - inter
            iou = inter / union
            cw_e = jnp.maximum(b1_x2, b2_x2) - jnp.minimum(b1_x1, b2_x1)
            ch_e = jnp.maximum(b1_y2, b2_y2) - jnp.minimum(b1_y1, b2_y1)
            c_area = cw_e * ch_e + 1e-16
            giou = iou - (c_area - union) / c_area

            g0 = jnp.where(valid, giou, 0.0)
            mu = lanesum(g0) / TOPK
            cdev = jnp.where(valid, giou - mu, 0.0)
            var = lanesum(cdev * cdev) / (TOPK - 1)
            thr = mu + var
            maskv = giou > thr

            ov[r, 0:16] = jnp.where(maskv, cx, 0.0)
            ov[r, 16:32] = jnp.where(maskv, cy, 0.0)
            ov[r, 32:48] = jnp.where(maskv, cw, 0.0)
            ov[r, 48:64] = jnp.where(maskv, ch, 0.0)
            ov[r, 64:80] = giou
            ov[r, 80:96] = jnp.where(maskv, 1.0, 0.0)
            ov[r, 96:112] = zf
            ov[r, 112:128] = zf

        pltpu.sync_copy(ov, out_hbm.at[pl.ds(base, rows_w)])

    return sc_fn


@functools.partial(jax.jit, static_argnames=("interpret",))
def kernel(p_boxes, target, interpret=False):
    n = p_boxes.shape[0]
    nt = target.shape[0]
    npad = ((n + GRP - 1) // GRP) * GRP
    ng = npad // GRP
    gpad = ((ng + 127) // 128) * 128
    tb = 64

    pc = p_boxes[:, 2:6]
    px = jnp.pad(pc[:, 0], (0, npad - n), constant_values=1e9)
    py = jnp.pad(pc[:, 1], (0, npad - n), constant_values=1e9)
    pref = jnp.stack([px, py], axis=0)
    tref = target[:, 2:6]

    gsel = pl.pallas_call(
        functools.partial(_tc1_body, npad=npad, tb=tb, gpad=gpad),
        grid=(nt // tb,),
        in_specs=[
            pl.BlockSpec((tb, 4), lambda i: (i, 0)),
            pl.BlockSpec((2, npad), lambda i: (0, 0)),
        ],
        out_specs=pl.BlockSpec((tb, 16), lambda i: (i, 0)),
        out_shape=jax.ShapeDtypeStruct((nt, 16), jnp.int32),
        compiler_params=pltpu.CompilerParams(
            dimension_semantics=("arbitrary",)),
        interpret=interpret,
    )(tref, pref)

    pxg = jnp.tile(px.reshape(ng, GRP), (32, 1))
    pyg = jnp.tile(py.reshape(ng, GRP), (32, 1))
    pxc, pyc = _sc_compact(nt, ng)(gsel, pxg, pyg)

    idx9 = pl.pallas_call(
        functools.partial(_tc2_body, tb=tb, npad=npad),
        grid=(nt // tb,),
        in_specs=[
            pl.BlockSpec((tb, 4), lambda i: (i, 0)),
            pl.BlockSpec((tb, TOPK * GRP), lambda i: (i, 0)),
            pl.BlockSpec((tb, TOPK * GRP), lambda i: (i, 0)),
            pl.BlockSpec((tb, 16), lambda i: (i, 0)),
        ],
        out_specs=pl.BlockSpec((tb, TOPK), lambda i: (i, 0)),
        out_shape=jax.ShapeDtypeStruct((nt, TOPK), jnp.int32),
        compiler_params=pltpu.CompilerParams(
            dimension_semantics=("arbitrary",)),
        interpret=interpret,
    )(tref, pxc.reshape(nt, TOPK * GRP), pyc.reshape(nt, TOPK * GRP), gsel)

    boxes128 = jnp.pad(pc, ((0, npad - n), (0, 124)))
    tgt128 = jnp.pad(tref, ((0, 0), (0, 124)))
    idx_flat = idx9.reshape(-1)

    out = _sc_epilogue(nt, npad)(idx_flat, boxes128, tgt128)

    pos = jnp.stack([out[:, 0:TOPK], out[:, 16:16 + TOPK],
                     out[:, 32:32 + TOPK], out[:, 48:48 + TOPK]], axis=-1)
    return pos, out[:, 64:64 + TOPK], out[:, 80:80 + TOPK].astype(bool)
